# baseline (speedup 1.0000x reference)
import functools

import jax
import jax.numpy as jnp
import numpy as np
from jax import lax
from jax.experimental import pallas as pl
from jax.experimental.pallas import tpu as pltpu
from jax.experimental.pallas import tpu_sc as plsc

LANES = 16


def _pos_encoding_np(length: int, d_model: int) -> np.ndarray:
    position = np.arange(0, length, dtype=np.float32)[:, None]
    i2 = np.arange(0, d_model, step=2).astype(np.float32)
    emb = np.zeros((length, d_model), dtype=np.float32)
    emb[:, 0::2] = np.sin(position / 10000 ** (i2 / d_model))
    emb[:, 1::2] = np.cos(position / 10000 ** (i2 / d_model))
    return emb


@functools.lru_cache(maxsize=None)
def _pos_const(length: int, d_model: int):
    return jnp.asarray(_pos_encoding_np(length, d_model))


def _sc_info():
    try:
        info = plsc.get_sparse_core_info()
        return info.num_cores, info.num_subcores
    except Exception:
        return 2, 16


@functools.lru_cache(maxsize=None)
def _build(B: int, L: int, D: int):
    NC, NS = _sc_info()
    NW = NC * NS
    assert L % NW == 0
    cols = L // NW
    CH = 32
    assert cols % CH == 0
    n_chunks_per_b = cols // CH
    nvec = D // LANES

    mesh = plsc.VectorSubcoreMesh(core_axis_name="c", subcore_axis_name="s")

    NBUF = 3
    chunks = [(b, c * CH) for b in range(B) for c in range(n_chunks_per_b)]
    N = len(chunks)

    @functools.partial(
        pl.kernel,
        mesh=mesh,
        out_type=jax.ShapeDtypeStruct((B, L, D), jnp.float32),
        scratch_types=[
            pltpu.VMEM((B, cols), jnp.int32),
            pltpu.VMEM((cols, D), jnp.float32),
            pltpu.VMEM((NBUF, CH, D), jnp.float32),
            pltpu.SemaphoreType.DMA,
            pltpu.SemaphoreType.DMA,
            pltpu.SemaphoreType.DMA,
            pltpu.SemaphoreType.DMA,
            pltpu.SemaphoreType.DMA,
            pltpu.SemaphoreType.DMA,
            pltpu.SemaphoreType.DMA,
            pltpu.SemaphoreType.DMA,
        ],
    )
    def k(x_hbm, table_hbm, pos_hbm, out_hbm, idx_v, pos_v, bufs,
          g0, g1, g2, w0, w1, w2, psem, isem):
        gsems = (g0, g1, g2)
        wsems = (w0, w1, w2)
        wid = lax.axis_index("s") * NC + lax.axis_index("c")
        l0 = wid * cols
        icps = [
            pltpu.async_copy(
                x_hbm.at[b, pl.ds(l0, cols)], idx_v.at[b],
                gsems[0] if b == 0 else isem)
            for b in range(B)
        ]
        pos_cp = pltpu.async_copy(pos_hbm.at[pl.ds(l0, cols)], pos_v, psem)

        def gather(i):
            b, off = chunks[i]
            s = i % NBUF
            return pltpu.async_copy(
                table_hbm.at[idx_v.at[b, pl.ds(off, CH)]], bufs.at[s],
                gsems[s]
            )

        g = [None] * N
        w = [None] * N
        icps[0].wait()
        g[0] = gather(0)
        for cp in icps[1:]:
            cp.wait()
        for i in range(N):
            s = i % NBUF
            b, off = chunks[i]
            if i + 1 < N:
                if i + 1 >= NBUF:
                    w[i + 1 - NBUF].wait()
                g[i + 1] = gather(i + 1)
            g[i].wait()
            if i == 0:
                pos_cp.wait()

            @plsc.parallel_loop(0, CH, 1, unroll=2)
            def row_body(r, off=off, s=s):
                for j in range(nvec):
                    sl = pl.ds(j * LANES, LANES)
                    plsc.addupdate(bufs.at[s, r, sl], pos_v[off + r, sl])
            w[i] = pltpu.async_copy(
                bufs.at[s], out_hbm.at[b, pl.ds(l0 + off, CH)], wsems[s]
            )
        for i in range(max(0, N - NBUF), N):
            w[i].wait()

    return k


def kernel(x, table):
    B, L = x.shape
    D = table.shape[1]
    pos = _pos_const(L, D)
    return _build(B, L, D)(x, table, pos)
